# Initial kernel scaffold; baseline (speedup 1.0000x reference)
import jax
import jax.numpy as jnp
from jax.experimental import pallas as pl


def kernel(src_feature_0, src_feature_1, src_embedding_0, src_embedding_1, src_y_0, src_y_1, src_graphlet_0, src_graphlet_1, W, b, attn_W, attn_b):
    raise NotImplementedError("write your pallas kernel here")


# speedup vs baseline: 1.0006x; 1.0006x over previous
import jax
import jax.numpy as jnp

KNN = 5


def kernel(src_feature_0, src_feature_1, src_embedding_0, src_embedding_1,
           src_y_0, src_y_1, src_graphlet_0, src_graphlet_1,
           W, b, attn_W, attn_b):
    n0 = src_embedding_0.shape[0]
    N = n0 + src_embedding_1.shape[0]
    emb = jnp.vstack([src_embedding_0, src_embedding_1])

    e = jax.lax.stop_gradient(emb)
    v = e / jnp.maximum(jnp.linalg.norm(e, axis=1, keepdims=True), 1e-12)
    adj = v @ v.T
    adj = adj.at[:n0, :n0].set(0.0)
    adj = adj.at[n0:, n0:].set(0.0)
    _, topk_idx = jax.lax.top_k(adj, KNN)
    sp_row = jnp.repeat(jnp.arange(N), KNN)
    sp_col = topk_idx.reshape(-1)

    sl = jnp.arange(N)
    row = jnp.concatenate([sp_row, sl])
    col = jnp.concatenate([sp_col, sl])
    deg = jnp.zeros((N,), dtype=jnp.float32).at[col].add(1.0)
    dinv = jnp.where(deg > 0, deg ** -0.5, 0.0)
    norm = dinv[row] * dinv[col]
    h = emb @ W.T
    updated = jnp.zeros((N, h.shape[1]), dtype=h.dtype).at[col].add(norm[:, None] * h[row]) + b

    stacked = jnp.stack([emb, updated], axis=1)
    logits = stacked @ attn_W.T + attn_b
    wts = jax.nn.softmax(logits, axis=1)
    updated_emb = jnp.sum(stacked * wts, axis=1)

    e0 = updated_emb[:n0]
    e1 = updated_emb[n0:]
    d = jnp.mean(e0, axis=0) - jnp.mean(e1, axis=0)
    loss = jnp.sum(d * d)
    return (loss, updated_emb)


# speedup vs baseline: 7.0442x; 7.0397x over previous
import functools

import jax
import jax.numpy as jnp
from jax.experimental import pallas as pl
from jax.experimental.pallas import tpu as pltpu

KNN = 5
HID = 128
RB = 256
NBLK = 8192 // RB


def _normalize_body(e_ref, v_ref):
    e = e_ref[...]
    ss = jnp.sum(e * e, axis=1, keepdims=True)
    nrm = jnp.sqrt(ss)
    v_ref[...] = e / jnp.maximum(nrm, 1e-12)


def _topk_body(vr_ref, vo_ref, idx_ref):
    i = pl.program_id(0)
    base = (1 - i // (NBLK // 2)) * 4096
    vr = vr_ref[...]
    vo = vo_ref[...]
    s = jax.lax.dot_general(vr, vo, (((1,), (1,)), ((), ())))
    cols = jax.lax.broadcasted_iota(jnp.int32, (RB, 4096), 1)
    rows = []
    for _ in range(KNN):
        m = jnp.max(s, axis=1, keepdims=True)
        hit = s >= m
        cand = jnp.where(hit, cols, 4096)
        idx_k = jnp.min(cand, axis=1)
        s = jnp.where(cols == idx_k[:, None], -3.0, s)
        rows.append((idx_k + base).reshape(1, RB))
    idx_ref[...] = jnp.concatenate(rows, axis=0)


def _hm_body(emb_ref, w_ref, cnt_ref, h_ref, m_ref, dinv_ref):
    emb = emb_ref[...]
    w = w_ref[...]
    h = jax.lax.dot_general(emb, w, (((1,), (1,)), ((), ())),
                            precision=jax.lax.Precision.HIGHEST)
    cnt = cnt_ref[0, :, 0:1] + cnt_ref[1, :, 0:1]
    deg = cnt + 1.0
    dinv = jax.lax.rsqrt(deg)
    m = h * dinv
    h_ref[...] = h
    dinv_ref[...] = dinv
    m_ref[0, :, :] = jax.lax.slice(m, (0, 0), (m.shape[0], 64))
    m_ref[1, :, :] = jax.lax.slice(m, (0, 64), (m.shape[0], 128))


def _final_body(emb_ref, h_ref, acc_ref, dinv_ref, b_ref, aw_ref, ab_ref,
                out_ref, loss_ref, sacc):
    j = pl.program_id(0)
    emb = emb_ref[...]
    h = h_ref[...]
    dinv = dinv_ref[...]
    accf = jnp.concatenate([acc_ref[0], acc_ref[1]], axis=1)
    updated = dinv * accf + (dinv * dinv) * h + b_ref[...]

    aw = aw_ref[...]
    ab = ab_ref[0, 0]
    l0 = jnp.sum(emb * aw, axis=1, keepdims=True) + ab
    l1 = jnp.sum(updated * aw, axis=1, keepdims=True) + ab
    mx = jnp.maximum(l0, l1)
    a0 = jnp.exp(l0 - mx)
    a1 = jnp.exp(l1 - mx)
    inv = 1.0 / (a0 + a1)
    out = (a0 * inv) * emb + (a1 * inv) * updated
    out_ref[...] = out

    @pl.when(j == 0)
    def _():
        sacc[...] = jnp.zeros_like(sacc)

    rs = jnp.sum(out, axis=0, keepdims=True)

    @pl.when(j < 8)
    def _():
        sacc[0:1, :] += rs

    @pl.when(j >= 8)
    def _():
        sacc[1:2, :] += rs

    @pl.when(j == 15)
    def _():
        d = (sacc[0:1, :] - sacc[1:2, :]) * (1.0 / 4096.0)
        loss_ref[...] = jnp.sum(d * d, axis=1, keepdims=True)


def kernel(src_feature_0, src_feature_1, src_embedding_0, src_embedding_1,
           src_y_0, src_y_1, src_graphlet_0, src_graphlet_1,
           W, b, attn_W, attn_b):
    emb = jnp.concatenate([src_embedding_0, src_embedding_1], axis=0)
    N = emb.shape[0]

    v = pl.pallas_call(
        _normalize_body,
        grid=(8,),
        in_specs=[pl.BlockSpec((1024, HID), lambda i: (i, 0))],
        out_specs=pl.BlockSpec((1024, HID), lambda i: (i, 0)),
        out_shape=jax.ShapeDtypeStruct((N, HID), jnp.float32),
    )(emb)

    idx_t = pl.pallas_call(
        _topk_body,
        grid=(NBLK,),
        in_specs=[
            pl.BlockSpec((RB, HID), lambda i: (i, 0)),
            pl.BlockSpec((4096, HID), lambda i: (1 - i // (NBLK // 2), 0)),
        ],
        out_specs=pl.BlockSpec((KNN, RB), lambda i: (0, i)),
        out_shape=jax.ShapeDtypeStruct((KNN, N), jnp.int32),
    )(v, v)

    cnt1 = jnp.zeros((N,), jnp.float32).at[idx_t.reshape(-1)].add(1.0)
    cnt_part = jnp.stack([cnt1, jnp.zeros_like(cnt1)])[:, :, None] * jnp.ones(
        (1, 1, 16), jnp.float32)

    h, m_split, dinv = pl.pallas_call(
        _hm_body,
        grid=(8,),
        in_specs=[
            pl.BlockSpec((1024, HID), lambda i: (i, 0)),
            pl.BlockSpec((HID, HID), lambda i: (0, 0)),
            pl.BlockSpec((2, 1024, 16), lambda i: (0, i, 0)),
        ],
        out_specs=[
            pl.BlockSpec((1024, HID), lambda i: (i, 0)),
            pl.BlockSpec((2, 1024, 64), lambda i: (0, i, 0)),
            pl.BlockSpec((1024, 1), lambda i: (i, 0)),
        ],
        out_shape=[
            jax.ShapeDtypeStruct((N, HID), jnp.float32),
            jax.ShapeDtypeStruct((2, N, 64), jnp.float32),
            jax.ShapeDtypeStruct((N, 1), jnp.float32),
        ],
    )(emb, W, cnt_part)

    m_full = jnp.concatenate([m_split[0], m_split[1]], axis=1)
    col = idx_t.T.reshape(-1)
    row = jnp.tile(jnp.arange(N)[:, None], (1, KNN)).reshape(-1)
    acc_full = jnp.zeros((N, HID), jnp.float32).at[col].add(m_full[row])
    acc = jnp.stack([acc_full[:, :64], acc_full[:, 64:]])

    updated_emb, loss = pl.pallas_call(
        _final_body,
        grid=(16,),
        in_specs=[
            pl.BlockSpec((512, HID), lambda j: (j, 0)),
            pl.BlockSpec((512, HID), lambda j: (j, 0)),
            pl.BlockSpec((2, 512, 64), lambda j: (0, j, 0)),
            pl.BlockSpec((512, 1), lambda j: (j, 0)),
            pl.BlockSpec((1, HID), lambda j: (0, 0)),
            pl.BlockSpec((1, HID), lambda j: (0, 0)),
            pl.BlockSpec((1, 1), lambda j: (0, 0)),
        ],
        out_specs=[
            pl.BlockSpec((512, HID), lambda j: (j, 0)),
            pl.BlockSpec((1, 1), lambda j: (0, 0)),
        ],
        out_shape=[
            jax.ShapeDtypeStruct((N, HID), jnp.float32),
            jax.ShapeDtypeStruct((1, 1), jnp.float32),
        ],
        scratch_shapes=[pltpu.VMEM((2, HID), jnp.float32)],
    )(emb, h, acc, dinv, b.reshape(1, HID), attn_W.reshape(1, HID),
      attn_b.reshape(1, 1))

    return (loss[0, 0], updated_emb)


# speedup vs baseline: 15.1232x; 2.1469x over previous
import jax
import jax.numpy as jnp
from jax import lax
from jax.experimental import pallas as pl
from jax.experimental.pallas import tpu as pltpu
from jax.experimental.pallas import tpu_sc as plsc

KNN = 5
HID = 128
RB = 256
NBLK = 8192 // RB


def _normalize_body(e_ref, v_ref):
    e = e_ref[...]
    ss = jnp.sum(e * e, axis=1, keepdims=True)
    nrm = jnp.sqrt(ss)
    v_ref[...] = e / jnp.maximum(nrm, 1e-12)


def _topk_body(vr_ref, vo_ref, idx_ref):
    i = pl.program_id(0)
    base = (1 - i // (NBLK // 2)) * 4096
    vr = vr_ref[...]
    vo = vo_ref[...]
    s = jax.lax.dot_general(vr, vo, (((1,), (1,)), ((), ())))
    cols = jax.lax.broadcasted_iota(jnp.int32, (RB, 4096), 1)
    rows = []
    for _ in range(KNN):
        m = jnp.max(s, axis=1, keepdims=True)
        hit = s >= m
        cand = jnp.where(hit, cols, 4096)
        idx_k = jnp.min(cand, axis=1)
        s = jnp.where(cols == idx_k[:, None], -3.0, s)
        rows.append((idx_k + base).reshape(1, RB))
    idx_ref[...] = jnp.concatenate(rows, axis=0)


def _hm_body(emb_ref, w_ref, cnt_ref, h_ref, m_ref, dinv_ref):
    emb = emb_ref[...]
    w = w_ref[...]
    h = jax.lax.dot_general(emb, w, (((1,), (1,)), ((), ())),
                            precision=jax.lax.Precision.HIGHEST)
    cnt = cnt_ref[0, :, 0:1] + cnt_ref[1, :, 0:1]
    deg = cnt + 1.0
    dinv = jax.lax.rsqrt(deg)
    h_ref[...] = h
    dinv_ref[...] = dinv
    m_ref[...] = h * dinv


def _final_body(emb_ref, h_ref, acc_ref, dinv_ref, b_ref, aw_ref, ab_ref,
                out_ref, loss_ref, sacc):
    j = pl.program_id(0)
    emb = emb_ref[...]
    h = h_ref[...]
    dinv = dinv_ref[...]
    accf = acc_ref[0] + acc_ref[1]
    updated = dinv * accf + (dinv * dinv) * h + b_ref[...]

    aw = aw_ref[...]
    ab = ab_ref[0, 0]
    l0 = jnp.sum(emb * aw, axis=1, keepdims=True) + ab
    l1 = jnp.sum(updated * aw, axis=1, keepdims=True) + ab
    mx = jnp.maximum(l0, l1)
    a0 = jnp.exp(l0 - mx)
    a1 = jnp.exp(l1 - mx)
    inv = 1.0 / (a0 + a1)
    out = (a0 * inv) * emb + (a1 * inv) * updated
    out_ref[...] = out

    @pl.when(j == 0)
    def _():
        sacc[...] = jnp.zeros_like(sacc)

    rs = jnp.sum(out, axis=0, keepdims=True)

    @pl.when(j < 8)
    def _():
        sacc[0:1, :] += rs

    @pl.when(j >= 8)
    def _():
        sacc[1:2, :] += rs

    @pl.when(j == 15)
    def _():
        d = (sacc[0:1, :] - sacc[1:2, :]) * (1.0 / 4096.0)
        loss_ref[...] = jnp.sum(d * d, axis=1, keepdims=True)


def _deg_body(idx_hbm, zeros_hbm, ones_hbm, out_hbm, idxv, onesv, acc_sh):
    q = lax.axis_index("c")
    t = lax.axis_index("s")
    base = q * 4096 + t * 256
    pltpu.sync_copy(zeros_hbm.at[pl.ds(t * 512, 512)],
                    acc_sh.at[pl.ds(t * 512, 512)])
    pltpu.sync_copy(ones_hbm, onesv)
    for k in range(KNN):
        for c in range(2):
            pltpu.sync_copy(
                idx_hbm.at[pl.ds(k * 8192 + base + c * 128, 128)],
                idxv.at[k * 2 + c])
    plsc.subcore_barrier()
    for k in range(KNN):
        for c in range(2):
            pltpu.sync_copy(onesv, acc_sh.at[idxv.at[k * 2 + c]], add=True)
    plsc.subcore_barrier()
    pltpu.sync_copy(acc_sh.at[pl.ds(t * 512, 512)],
                    out_hbm.at[q, pl.ds(t * 512, 512)])


def _msg_body(idx_hbm, m_hbm, zeros_hbm, out_hbm, idxv, mv, acc_sh):
    q = lax.axis_index("c")
    t = lax.axis_index("s")
    base = q * 4096 + t * 256
    pltpu.sync_copy(zeros_hbm.at[pl.ds(t * 512, 512)],
                    acc_sh.at[pl.ds(t * 512, 512)])
    pltpu.sync_copy(m_hbm.at[pl.ds(base, 256)], mv)
    for k in range(KNN):
        for c in range(2):
            pltpu.sync_copy(
                idx_hbm.at[pl.ds(k * 8192 + base + c * 128, 128)],
                idxv.at[k * 2 + c])
    plsc.subcore_barrier()
    for k in range(KNN):
        for c in range(2):
            pltpu.sync_copy(mv.at[pl.ds(c * 128, 128)],
                            acc_sh.at[idxv.at[k * 2 + c]], add=True)
    plsc.subcore_barrier()
    pltpu.sync_copy(acc_sh.at[pl.ds(t * 512, 512)],
                    out_hbm.at[q, pl.ds(t * 512, 512)])


def kernel(src_feature_0, src_feature_1, src_embedding_0, src_embedding_1,
           src_y_0, src_y_1, src_graphlet_0, src_graphlet_1,
           W, b, attn_W, attn_b):
    emb = jnp.concatenate([src_embedding_0, src_embedding_1], axis=0)
    N = emb.shape[0]

    v = pl.pallas_call(
        _normalize_body,
        grid=(8,),
        in_specs=[pl.BlockSpec((1024, HID), lambda i: (i, 0))],
        out_specs=pl.BlockSpec((1024, HID), lambda i: (i, 0)),
        out_shape=jax.ShapeDtypeStruct((N, HID), jnp.float32),
    )(emb)

    idx_t = pl.pallas_call(
        _topk_body,
        grid=(NBLK,),
        in_specs=[
            pl.BlockSpec((RB, HID), lambda i: (i, 0)),
            pl.BlockSpec((4096, HID), lambda i: (1 - i // (NBLK // 2), 0)),
        ],
        out_specs=pl.BlockSpec((KNN, RB), lambda i: (0, i)),
        out_shape=jax.ShapeDtypeStruct((KNN, N), jnp.int32),
    )(v, v)

    mesh = plsc.VectorSubcoreMesh(core_axis_name="c", subcore_axis_name="s")
    idx_flat = idx_t.reshape(-1)
    zeros128 = jnp.zeros((N, HID), jnp.float32)
    ones128 = jnp.ones((128, HID), jnp.float32)
    cnt_part = pl.kernel(
        _deg_body,
        out_type=jax.ShapeDtypeStruct((2, N, HID), jnp.float32),
        mesh=mesh,
        scratch_types=[
            pltpu.VMEM((KNN * 2, 128), jnp.int32),
            pltpu.VMEM((128, HID), jnp.float32),
            pltpu.VMEM_SHARED((N, HID), jnp.float32),
        ],
    )(idx_flat, zeros128, ones128)

    h, m, dinv = pl.pallas_call(
        _hm_body,
        grid=(8,),
        in_specs=[
            pl.BlockSpec((1024, HID), lambda i: (i, 0)),
            pl.BlockSpec((HID, HID), lambda i: (0, 0)),
            pl.BlockSpec((2, 1024, HID), lambda i: (0, i, 0)),
        ],
        out_specs=[
            pl.BlockSpec((1024, HID), lambda i: (i, 0)),
            pl.BlockSpec((1024, HID), lambda i: (i, 0)),
            pl.BlockSpec((1024, 1), lambda i: (i, 0)),
        ],
        out_shape=[
            jax.ShapeDtypeStruct((N, HID), jnp.float32),
            jax.ShapeDtypeStruct((N, HID), jnp.float32),
            jax.ShapeDtypeStruct((N, 1), jnp.float32),
        ],
    )(emb, W, cnt_part)

    acc = pl.kernel(
        _msg_body,
        out_type=jax.ShapeDtypeStruct((2, N, HID), jnp.float32),
        mesh=mesh,
        scratch_types=[
            pltpu.VMEM((KNN * 2, 128), jnp.int32),
            pltpu.VMEM((256, HID), jnp.float32),
            pltpu.VMEM_SHARED((N, HID), jnp.float32),
        ],
    )(idx_flat, m, zeros128)

    updated_emb, loss = pl.pallas_call(
        _final_body,
        grid=(16,),
        in_specs=[
            pl.BlockSpec((512, HID), lambda j: (j, 0)),
            pl.BlockSpec((512, HID), lambda j: (j, 0)),
            pl.BlockSpec((2, 512, HID), lambda j: (0, j, 0)),
            pl.BlockSpec((512, 1), lambda j: (j, 0)),
            pl.BlockSpec((1, HID), lambda j: (0, 0)),
            pl.BlockSpec((1, HID), lambda j: (0, 0)),
            pl.BlockSpec((1, 1), lambda j: (0, 0)),
        ],
        out_specs=[
            pl.BlockSpec((512, HID), lambda j: (j, 0)),
            pl.BlockSpec((1, 1), lambda j: (0, 0)),
        ],
        out_shape=[
            jax.ShapeDtypeStruct((N, HID), jnp.float32),
            jax.ShapeDtypeStruct((1, 1), jnp.float32),
        ],
        scratch_shapes=[pltpu.VMEM((2, HID), jnp.float32)],
    )(emb, h, acc, dinv, b.reshape(1, HID), attn_W.reshape(1, HID),
      attn_b.reshape(1, 1))

    return (loss[0, 0], updated_emb)
